# baseline (speedup 1.0000x reference)
import functools

import jax
import jax.numpy as jnp
from jax import lax
from jax.experimental import pallas as pl
from jax.experimental.pallas import tpu as pltpu
from jax.experimental.pallas import tpu_sc as plsc

N_NODES = 10000
N_EDGES = 160000
D_IN = 256
D_OUT = 256
HALF = 128
HALF_N = N_NODES // 2

NC = 2
NS = 16
LANES = 16

E_PAD = 163840
EPT = E_PAD // NS
SEG = 2048
NSEG = EPT // SEG
CHUNK = 32
MAXC = SEG // CHUNK
FLT_CAP = SEG + 2 * LANES
RPT = 312


def _sc_agg_body(x_hbm, pk_hbm, ev_hbm, out_hbm,
                 pk_seg, ev_seg, flt_pk, flt_ev, gidx_r, dstl_r,
                 rows_a, rows_b, lo_a, lo_b, hi_a, hi_b,
                 acc_lo, acc_hi, gsem, ssem, isem):
    c = lax.axis_index("c")
    s = lax.axis_index("s")
    rows = (rows_a, rows_b)
    los = (lo_a, lo_b)
    his = (hi_a, hi_b)
    tb = pl.multiple_of(s * EPT, 8)

    def issue_seg(sg):
        lo = tb + sg * SEG
        pltpu.async_copy(pk_hbm.at[c, pl.ds(lo, SEG)], pk_seg.at[sg % 2], isem)
        pltpu.async_copy(ev_hbm.at[pl.ds(lo, SEG)], ev_seg.at[sg % 2], isem)

    def wait_seg(sg):
        lo = tb + sg * SEG
        pltpu.make_async_copy(pk_hbm.at[c, pl.ds(lo, SEG)],
                              pk_seg.at[sg % 2], isem).wait()
        pltpu.make_async_copy(ev_hbm.at[pl.ds(lo, SEG)],
                              ev_seg.at[sg % 2], isem).wait()

    issue_seg(0)

    zero16 = jnp.zeros((LANES,), jnp.float32)

    def zrow(r, carry):
        for p in range(HALF // LANES):
            lo_a[r, pl.ds(p * LANES, LANES)] = zero16
        return carry

    lax.fori_loop(0, CHUNK, zrow, 0)
    rstart = pl.multiple_of(s * RPT, 8)
    for acc in (acc_lo, acc_hi):
        for k in range(9):
            pltpu.sync_copy(lo_a, acc.at[pl.ds(rstart + k * CHUNK, CHUNK)])
        pltpu.sync_copy(lo_a.at[pl.ds(0, RPT - 9 * CHUNK)],
                        acc.at[pl.ds(rstart + 9 * CHUNK, RPT - 9 * CHUNK)])

        @pl.when(s == NS - 1)
        def _():
            pltpu.sync_copy(lo_a.at[pl.ds(0, HALF_N - NS * RPT)],
                            acc.at[pl.ds(NS * RPT, HALF_N - NS * RPT)])

    plsc.subcore_barrier()

    def prep_gidx(j, pj):
        w0 = flt_pk[pl.ds(j * CHUNK, LANES)]
        w1 = flt_pk[pl.ds(j * CHUNK + LANES, LANES)]
        gidx_r[pj, pl.ds(0, LANES)] = w0 & 0xFFFF
        gidx_r[pj, pl.ds(LANES, LANES)] = w1 & 0xFFFF

    def prep_dstl(j, pj):
        w0 = flt_pk[pl.ds(j * CHUNK, LANES)]
        w1 = flt_pk[pl.ds(j * CHUNK + LANES, LANES)]
        dstl_r[pj, pl.ds(0, LANES)] = w0 >> 16
        dstl_r[pj, pl.ds(LANES, LANES)] = w1 >> 16

    def start_gather(pj):
        pltpu.async_copy(x_hbm.at[gidx_r.at[pj]], rows[pj], gsem)

    def wait_gather(pj):
        pltpu.make_async_copy(x_hbm.at[gidx_r.at[pj]], rows[pj], gsem).wait()

    def start_scatter(pj):
        pltpu.async_copy(los[pj], acc_lo.at[dstl_r.at[pj]], ssem, add=True)
        pltpu.async_copy(his[pj], acc_hi.at[dstl_r.at[pj]], ssem, add=True)

    def wait_scatter_pair():
        pltpu.make_async_copy(lo_a, acc_lo.at[dstl_r.at[0]], ssem).wait()
        pltpu.make_async_copy(hi_a, acc_hi.at[dstl_r.at[0]], ssem).wait()

    def scale_split(j, pj):
        src_rows = rows[pj]
        dst_lo = los[pj]
        dst_hi = his[pj]

        def srow(gg, carry):
            evvec = flt_ev[pl.ds(j * CHUNK + gg * LANES, LANES)]
            for l in range(LANES):
                e = evvec[l]
                g = gg * LANES + l
                for p in range(HALF // LANES):
                    sl = pl.ds(p * LANES, LANES)
                    dst_lo[g, sl] = src_rows[g, sl] * e
                for p in range(HALF // LANES):
                    sl = pl.ds(p * LANES, LANES)
                    sh = pl.ds(HALF + p * LANES, LANES)
                    dst_hi[g, sl] = src_rows[g, sh] * e
            return carry

        lax.fori_loop(0, CHUNK // LANES, srow, 0)

    def seg_body(sg, carry):
        wait_seg(sg)

        @pl.when(sg + 1 < NSEG)
        def _():
            issue_seg(sg + 1)

        sg2 = sg % 2
        zeros_i = jnp.zeros((LANES,), jnp.int32)

        def fbody(i, ptr):
            w = pk_seg[sg2, pl.ds(i * LANES, LANES)]
            ev = ev_seg[sg2, pl.ds(i * LANES, LANES)]
            d = w >> 16
            mi = ((d | (HALF_N - 1 - d)) >> 31) + 1
            for l in range(LANES):
                flt_pk[pl.ds(ptr, LANES)] = zeros_i + w[l]
                flt_ev[pl.ds(ptr, LANES)] = zero16 + ev[l]
                ptr = ptr + mi[l]
            return ptr

        nloc = lax.fori_loop(0, SEG // LANES, fbody, jnp.int32(0))
        for t in range(CHUNK // LANES):
            flt_pk[pl.ds(nloc + t * LANES, LANES)] = zeros_i
            flt_ev[pl.ds(nloc + t * LANES, LANES)] = zero16
        nch = (nloc + CHUNK - 1) // CHUNK

        @pl.when(nch > 0)
        def _():
            prep_gidx(0, 0)
            start_gather(0)

        def pair_body(jo, carry2):
            for jjj in range(2):
                j = jo * 2 + jjj
                pj = jjj
                npj = 1 - jjj

                @pl.when(j + 1 < nch)
                def _():
                    prep_gidx(j + 1, npj)
                    start_gather(npj)

                @pl.when((j >= 2) & (j <= nch + 1))
                def _():
                    wait_scatter_pair()

                @pl.when(j < nch)
                def _():
                    wait_gather(pj)
                    prep_dstl(j, pj)
                    scale_split(j, pj)
                    start_scatter(pj)
            return carry2

        lax.fori_loop(0, (MAXC + 2) // 2, pair_body, 0)
        return carry

    lax.fori_loop(0, NSEG, seg_body, 0)
    plsc.subcore_barrier()

    for h, acc in enumerate((acc_lo, acc_hi)):
        pltpu.sync_copy(acc.at[pl.ds(rstart, RPT)],
                        out_hbm.at[c, h, pl.ds(rstart, RPT)])

        @pl.when(s == NS - 1)
        def _():
            pltpu.sync_copy(acc.at[pl.ds(NS * RPT, HALF_N - NS * RPT)],
                            out_hbm.at[c, h, pl.ds(NS * RPT, HALF_N - NS * RPT)])


_sc_agg = functools.partial(
    pl.kernel,
    out_type=jax.ShapeDtypeStruct((NC, 2, HALF_N, HALF), jnp.float32),
    mesh=plsc.VectorSubcoreMesh(core_axis_name="c", subcore_axis_name="s"),
    scratch_types=[
        pltpu.VMEM((2, SEG), jnp.int32),
        pltpu.VMEM((2, SEG), jnp.float32),
        pltpu.VMEM((FLT_CAP,), jnp.int32),
        pltpu.VMEM((FLT_CAP,), jnp.float32),
        pltpu.VMEM((2, CHUNK), jnp.int32),
        pltpu.VMEM((2, CHUNK), jnp.int32),
        pltpu.VMEM((CHUNK, D_IN), jnp.float32),
        pltpu.VMEM((CHUNK, D_IN), jnp.float32),
        pltpu.VMEM((CHUNK, HALF), jnp.float32),
        pltpu.VMEM((CHUNK, HALF), jnp.float32),
        pltpu.VMEM((CHUNK, HALF), jnp.float32),
        pltpu.VMEM((CHUNK, HALF), jnp.float32),
        pltpu.VMEM_SHARED((HALF_N, HALF), jnp.float32),
        pltpu.VMEM_SHARED((HALF_N, HALF), jnp.float32),
        pltpu.SemaphoreType.DMA,
        pltpu.SemaphoreType.DMA,
        pltpu.SemaphoreType.DMA,
    ],
)(_sc_agg_body)


def _mm_body(a_ref, w_ref, o_ref):
    o_ref[...] = jnp.maximum(
        jnp.dot(a_ref[...], w_ref[...],
                preferred_element_type=jnp.float32,
                precision=lax.Precision.HIGHEST), 0.0)


M_BLK = 1000


def _mm_relu(agg, w):
    return pl.pallas_call(
        _mm_body,
        grid=(N_NODES // M_BLK,),
        in_specs=[
            pl.BlockSpec((M_BLK, D_IN), lambda i: (i, 0)),
            pl.BlockSpec((D_IN, D_OUT), lambda i: (0, 0)),
        ],
        out_specs=pl.BlockSpec((M_BLK, D_OUT), lambda i: (i, 0)),
        out_shape=jax.ShapeDtypeStruct((N_NODES, D_OUT), jnp.float32),
    )(agg, w)


def kernel(x, edge_index, edge_values, W):
    pad = E_PAD - N_EDGES
    src = jnp.pad(edge_index[1], (0, pad))
    dst = jnp.pad(edge_index[0], (0, pad), constant_values=-8000)
    ev = jnp.pad(edge_values, (0, pad))
    pk = jnp.stack([src | (dst << 16), src | ((dst - HALF_N) << 16)])
    out4 = _sc_agg(x, pk, ev)
    agg = jnp.concatenate([out4[:, 0], out4[:, 1]], axis=-1)
    return _mm_relu(agg.reshape(N_NODES, D_IN), W)


# speedup vs baseline: 1.4047x; 1.4047x over previous
import functools

import jax
import jax.numpy as jnp
from jax import lax
from jax.experimental import pallas as pl
from jax.experimental.pallas import tpu as pltpu
from jax.experimental.pallas import tpu_sc as plsc

N_NODES = 10000
N_EDGES = 160000
D_IN = 256
D_OUT = 256
HALF = 128

NC = 2
NS = 16
LANES = 16

CHUNK = 128
N_CHUNKS = N_EDGES // CHUNK
CHUNKS_PER_TILE = -(-N_CHUNKS // NS)
ROWS_MAIN = 624


def _sc_agg_body(x2_hbm, ei_hbm, ev_hbm, out_hbm,
                 src_v, dst_v, ev_v, rows_v, acc_sh, sem):
    c = lax.axis_index("c")
    s = lax.axis_index("s")

    zero16 = jnp.zeros((LANES,), jnp.float32)

    def zrow(r, carry):
        for p in range(HALF // LANES):
            rows_v[r, pl.ds(p * LANES, LANES)] = zero16
        return carry

    lax.fori_loop(0, CHUNK, zrow, 0)
    start = pl.multiple_of(s * ROWS_MAIN, 8)
    for k in range(4):
        pltpu.sync_copy(rows_v, acc_sh.at[pl.ds(start + k * CHUNK, CHUNK)])
    pltpu.sync_copy(rows_v.at[pl.ds(0, ROWS_MAIN - 4 * CHUNK)],
                    acc_sh.at[pl.ds(start + 4 * CHUNK, ROWS_MAIN - 4 * CHUNK)])

    @pl.when(s == NS - 1)
    def _():
        pltpu.sync_copy(rows_v.at[pl.ds(0, N_NODES - NS * ROWS_MAIN)],
                        acc_sh.at[pl.ds(NS * ROWS_MAIN, N_NODES - NS * ROWS_MAIN)])

    plsc.subcore_barrier()

    def chunk_body(i, carry):
        cid = i * NS + s

        @pl.when(cid < N_CHUNKS)
        def _():
            base = cid * CHUNK
            pltpu.sync_copy(ei_hbm.at[1, pl.ds(base, CHUNK)], src_v)
            pltpu.sync_copy(ei_hbm.at[0, pl.ds(base, CHUNK)], dst_v)
            pltpu.sync_copy(ev_hbm.at[pl.ds(base, CHUNK)], ev_v)
            for p in range(CHUNK // LANES):
                sl = pl.ds(p * LANES, LANES)
                src_v[sl] = src_v[sl] * 2 + c
            pltpu.async_copy(x2_hbm.at[src_v], rows_v, sem).wait()

            def srow(gg, inner):
                evvec = ev_v[pl.ds(gg * LANES, LANES)]
                for l in range(LANES):
                    e = evvec[l]
                    g = gg * LANES + l
                    for p in range(HALF // LANES):
                        sl = pl.ds(p * LANES, LANES)
                        rows_v[g, sl] = rows_v[g, sl] * e
                return inner

            lax.fori_loop(0, CHUNK // LANES, srow, 0)
            pltpu.sync_copy(rows_v, acc_sh.at[dst_v], add=True)

        return carry

    lax.fori_loop(0, CHUNKS_PER_TILE, chunk_body, 0)
    plsc.subcore_barrier()
    pltpu.sync_copy(acc_sh.at[pl.ds(start, ROWS_MAIN)],
                    out_hbm.at[c, pl.ds(start, ROWS_MAIN)])

    @pl.when(s == NS - 1)
    def _():
        pltpu.sync_copy(
            acc_sh.at[pl.ds(NS * ROWS_MAIN, N_NODES - NS * ROWS_MAIN)],
            out_hbm.at[c, pl.ds(NS * ROWS_MAIN, N_NODES - NS * ROWS_MAIN)])


_sc_agg = functools.partial(
    pl.kernel,
    out_type=jax.ShapeDtypeStruct((NC, N_NODES, HALF), jnp.float32),
    mesh=plsc.VectorSubcoreMesh(core_axis_name="c", subcore_axis_name="s"),
    scratch_types=[
        pltpu.VMEM((CHUNK,), jnp.int32),
        pltpu.VMEM((CHUNK,), jnp.int32),
        pltpu.VMEM((CHUNK,), jnp.float32),
        pltpu.VMEM((CHUNK, HALF), jnp.float32),
        pltpu.VMEM_SHARED((N_NODES, HALF), jnp.float32),
        pltpu.SemaphoreType.DMA,
    ],
)(_sc_agg_body)


def _mm_body(a0_ref, a1_ref, w0_ref, w1_ref, o_ref):
    acc = jnp.dot(a0_ref[...], w0_ref[...],
                  preferred_element_type=jnp.float32,
                  precision=lax.Precision.HIGHEST)
    acc = acc + jnp.dot(a1_ref[...], w1_ref[...],
                        preferred_element_type=jnp.float32,
                        precision=lax.Precision.HIGHEST)
    o_ref[...] = jnp.maximum(acc, 0.0)


M_BLK = 1000


def _mm_relu(agg2, w):
    return pl.pallas_call(
        _mm_body,
        grid=(N_NODES // M_BLK,),
        in_specs=[
            pl.BlockSpec((M_BLK, HALF), lambda i: (i, 0)),
            pl.BlockSpec((M_BLK, HALF), lambda i: (i, 0)),
            pl.BlockSpec((HALF, D_OUT), lambda i: (0, 0)),
            pl.BlockSpec((HALF, D_OUT), lambda i: (0, 0)),
        ],
        out_specs=pl.BlockSpec((M_BLK, D_OUT), lambda i: (i, 0)),
        out_shape=jax.ShapeDtypeStruct((N_NODES, D_OUT), jnp.float32),
    )(agg2[0], agg2[1], w[:HALF], w[HALF:])


def kernel(x, edge_index, edge_values, W):
    x2 = x.reshape(2 * N_NODES, HALF)
    agg2 = _sc_agg(x2, edge_index, edge_values)
    return _mm_relu(agg2, W)


# speedup vs baseline: 2.5232x; 1.7962x over previous
import functools

import jax
import jax.numpy as jnp
from jax import lax
from jax.experimental import pallas as pl
from jax.experimental.pallas import tpu as pltpu
from jax.experimental.pallas import tpu_sc as plsc

N_NODES = 10000
N_EDGES = 160000
D_IN = 256
D_OUT = 256
HALF = 128

NC = 2
NS = 16
LANES = 16

CHUNK = 128
N_CHUNKS = N_EDGES // CHUNK
CHUNKS_PER_TILE = -(-N_CHUNKS // NS)
ROWS_MAIN = 624


def _sc_agg_body(x2_hbm, ei_hbm, ev_hbm, out_hbm,
                 src_v, dst_v, ev_v, rows_a, rows_b, acc_sh, gsem, isem):
    c = lax.axis_index("c")
    s = lax.axis_index("s")
    bufs = (rows_a, rows_b)

    zero16 = jnp.zeros((LANES,), jnp.float32)

    def zrow(r, carry):
        for p in range(HALF // LANES):
            rows_a[r, pl.ds(p * LANES, LANES)] = zero16
        return carry

    lax.fori_loop(0, CHUNK, zrow, 0)
    start = pl.multiple_of(s * ROWS_MAIN, 8)
    for k in range(4):
        pltpu.sync_copy(rows_a, acc_sh.at[pl.ds(start + k * CHUNK, CHUNK)])
    pltpu.sync_copy(rows_a.at[pl.ds(0, ROWS_MAIN - 4 * CHUNK)],
                    acc_sh.at[pl.ds(start + 4 * CHUNK, ROWS_MAIN - 4 * CHUNK)])

    @pl.when(s == NS - 1)
    def _():
        pltpu.sync_copy(rows_a.at[pl.ds(0, N_NODES - NS * ROWS_MAIN)],
                        acc_sh.at[pl.ds(NS * ROWS_MAIN, N_NODES - NS * ROWS_MAIN)])

    plsc.subcore_barrier()

    def issue_idx(i, sl):
        base = (i * NS + s) * CHUNK
        pltpu.async_copy(ei_hbm.at[1, pl.ds(base, CHUNK)], src_v.at[sl], isem)
        pltpu.async_copy(ei_hbm.at[0, pl.ds(base, CHUNK)], dst_v.at[sl], isem)
        pltpu.async_copy(ev_hbm.at[pl.ds(base, CHUNK)], ev_v.at[sl], isem)

    def wait_idx(i, sl):
        base = (i * NS + s) * CHUNK
        pltpu.make_async_copy(ei_hbm.at[1, pl.ds(base, CHUNK)],
                              src_v.at[sl], isem).wait()
        pltpu.make_async_copy(ei_hbm.at[0, pl.ds(base, CHUNK)],
                              dst_v.at[sl], isem).wait()
        pltpu.make_async_copy(ev_hbm.at[pl.ds(base, CHUNK)],
                              ev_v.at[sl], isem).wait()

    def xform(sl):
        for p in range(CHUNK // LANES):
            d = pl.ds(p * LANES, LANES)
            src_v[sl, d] = src_v[sl, d] * 2 + c

    def issue_gather(sl, buf):
        pltpu.async_copy(x2_hbm.at[src_v.at[sl]], buf, gsem)

    def wait_gather(sl, buf):
        pltpu.make_async_copy(x2_hbm.at[src_v.at[sl]], buf, gsem).wait()

    def scale(sl, buf):
        def srow(gg, inner):
            evvec = ev_v[sl, pl.ds(gg * LANES, LANES)]
            for l in range(LANES):
                e = evvec[l]
                g = gg * LANES + l
                for p in range(HALF // LANES):
                    d = pl.ds(p * LANES, LANES)
                    buf[g, d] = buf[g, d] * e
            return inner

        lax.fori_loop(0, CHUNK // LANES, srow, 0)

    issue_idx(0, 0)
    wait_idx(0, 0)
    xform(0)
    issue_gather(0, rows_a)

    def pair_body(jo, carry):
        for jj in range(2):
            i = jo * 2 + jj
            buf = bufs[jj]
            other = bufs[1 - jj]
            valid = i * NS + s < N_CHUNKS
            nxt = (i + 1) * NS + s < N_CHUNKS

            @pl.when(nxt)
            def _():
                issue_idx(i + 1, 1 - jj)

            @pl.when(valid)
            def _():
                wait_gather(jj, buf)

            @pl.when(nxt)
            def _():
                wait_idx(i + 1, 1 - jj)
                xform(1 - jj)
                issue_gather(1 - jj, other)

            @pl.when(valid)
            def _():
                scale(jj, buf)
                pltpu.sync_copy(buf, acc_sh.at[dst_v.at[jj]], add=True)
        return carry

    lax.fori_loop(0, (CHUNKS_PER_TILE + 1) // 2, pair_body, 0)
    plsc.subcore_barrier()
    pltpu.sync_copy(acc_sh.at[pl.ds(start, ROWS_MAIN)],
                    out_hbm.at[c, pl.ds(start, ROWS_MAIN)])

    @pl.when(s == NS - 1)
    def _():
        pltpu.sync_copy(
            acc_sh.at[pl.ds(NS * ROWS_MAIN, N_NODES - NS * ROWS_MAIN)],
            out_hbm.at[c, pl.ds(NS * ROWS_MAIN, N_NODES - NS * ROWS_MAIN)])


_sc_agg = functools.partial(
    pl.kernel,
    out_type=jax.ShapeDtypeStruct((NC, N_NODES, HALF), jnp.float32),
    mesh=plsc.VectorSubcoreMesh(core_axis_name="c", subcore_axis_name="s"),
    scratch_types=[
        pltpu.VMEM((2, CHUNK), jnp.int32),
        pltpu.VMEM((2, CHUNK), jnp.int32),
        pltpu.VMEM((2, CHUNK), jnp.float32),
        pltpu.VMEM((CHUNK, HALF), jnp.float32),
        pltpu.VMEM((CHUNK, HALF), jnp.float32),
        pltpu.VMEM_SHARED((N_NODES, HALF), jnp.float32),
        pltpu.SemaphoreType.DMA,
        pltpu.SemaphoreType.DMA,
    ],
)(_sc_agg_body)


def _mm_body(a0_ref, a1_ref, w0_ref, w1_ref, o_ref):
    acc = jnp.dot(a0_ref[...], w0_ref[...],
                  preferred_element_type=jnp.float32,
                  precision=lax.Precision.HIGHEST)
    acc = acc + jnp.dot(a1_ref[...], w1_ref[...],
                        preferred_element_type=jnp.float32,
                        precision=lax.Precision.HIGHEST)
    o_ref[...] = jnp.maximum(acc, 0.0)


M_BLK = 1000


def _mm_relu(agg2, w):
    return pl.pallas_call(
        _mm_body,
        grid=(N_NODES // M_BLK,),
        in_specs=[
            pl.BlockSpec((M_BLK, HALF), lambda i: (i, 0)),
            pl.BlockSpec((M_BLK, HALF), lambda i: (i, 0)),
            pl.BlockSpec((HALF, D_OUT), lambda i: (0, 0)),
            pl.BlockSpec((HALF, D_OUT), lambda i: (0, 0)),
        ],
        out_specs=pl.BlockSpec((M_BLK, D_OUT), lambda i: (i, 0)),
        out_shape=jax.ShapeDtypeStruct((N_NODES, D_OUT), jnp.float32),
    )(agg2[0], agg2[1], w[:HALF], w[HALF:])


def kernel(x, edge_index, edge_values, W):
    x2 = x.reshape(2 * N_NODES, HALF)
    agg2 = _sc_agg(x2, edge_index, edge_values)
    return _mm_relu(agg2, W)


# speedup vs baseline: 2.5291x; 1.0023x over previous
import functools

import jax
import jax.numpy as jnp
from jax import lax
from jax.experimental import pallas as pl
from jax.experimental.pallas import tpu as pltpu
from jax.experimental.pallas import tpu_sc as plsc

N_NODES = 10000
N_EDGES = 160000
D_IN = 256
D_OUT = 256
HALF = 128

NC = 2
NS = 16
LANES = 16

CHUNK = 128
N_CHUNKS = N_EDGES // CHUNK
CHUNKS_PER_TILE = -(-N_CHUNKS // NS)
ROWS_MAIN = 624


def _sc_agg_body(x2_hbm, ei_hbm, ev_hbm, out_hbm,
                 src_v, dst_v, ev_v, rows_a, rows_b, rows_c, acc_sh,
                 gsem, isem):
    c = lax.axis_index("c")
    s = lax.axis_index("s")
    bufs = (rows_a, rows_b, rows_c)

    zero16 = jnp.zeros((LANES,), jnp.float32)

    def zrow(r, carry):
        for p in range(HALF // LANES):
            rows_a[r, pl.ds(p * LANES, LANES)] = zero16
        return carry

    lax.fori_loop(0, CHUNK, zrow, 0)
    start = pl.multiple_of(s * ROWS_MAIN, 8)
    for k in range(4):
        pltpu.sync_copy(rows_a, acc_sh.at[pl.ds(start + k * CHUNK, CHUNK)])
    pltpu.sync_copy(rows_a.at[pl.ds(0, ROWS_MAIN - 4 * CHUNK)],
                    acc_sh.at[pl.ds(start + 4 * CHUNK, ROWS_MAIN - 4 * CHUNK)])

    @pl.when(s == NS - 1)
    def _():
        pltpu.sync_copy(rows_a.at[pl.ds(0, N_NODES - NS * ROWS_MAIN)],
                        acc_sh.at[pl.ds(NS * ROWS_MAIN, N_NODES - NS * ROWS_MAIN)])

    plsc.subcore_barrier()

    def issue_idx(i, sl):
        base = (i * NS + s) * CHUNK
        pltpu.async_copy(ei_hbm.at[1, pl.ds(base, CHUNK)], src_v.at[sl], isem)
        pltpu.async_copy(ei_hbm.at[0, pl.ds(base, CHUNK)], dst_v.at[sl], isem)
        pltpu.async_copy(ev_hbm.at[pl.ds(base, CHUNK)], ev_v.at[sl], isem)

    def wait_idx(i, sl):
        base = (i * NS + s) * CHUNK
        pltpu.make_async_copy(ei_hbm.at[1, pl.ds(base, CHUNK)],
                              src_v.at[sl], isem).wait()
        pltpu.make_async_copy(ei_hbm.at[0, pl.ds(base, CHUNK)],
                              dst_v.at[sl], isem).wait()
        pltpu.make_async_copy(ev_hbm.at[pl.ds(base, CHUNK)],
                              ev_v.at[sl], isem).wait()

    def xform(sl):
        for p in range(CHUNK // LANES):
            d = pl.ds(p * LANES, LANES)
            src_v[sl, d] = src_v[sl, d] * 2 + c

    def issue_gather(sl, buf):
        pltpu.async_copy(x2_hbm.at[src_v.at[sl]], buf, gsem)

    def wait_gather(sl, buf):
        pltpu.make_async_copy(x2_hbm.at[src_v.at[sl]], buf, gsem).wait()

    def scale(sl, buf):
        def srow(gg, inner):
            evvec = ev_v[sl, pl.ds(gg * LANES, LANES)]
            for l in range(LANES):
                e = evvec[l]
                g = gg * LANES + l
                for p in range(HALF // LANES):
                    d = pl.ds(p * LANES, LANES)
                    buf[g, d] = buf[g, d] * e
            return inner

        lax.fori_loop(0, CHUNK // LANES, srow, 0)

    issue_idx(0, 0)
    issue_idx(1, 1)
    issue_idx(2, 2)
    wait_idx(0, 0)
    xform(0)
    issue_gather(0, rows_a)
    wait_idx(1, 1)
    xform(1)
    issue_gather(1, rows_b)

    def ring_body(jo, carry):
        for jj in range(3):
            i = jo * 3 + jj
            buf = bufs[jj]
            nj = (jj + 2) % 3
            valid = i * NS + s < N_CHUNKS

            @pl.when(valid)
            def _():
                wait_gather(jj, buf)

            @pl.when((i + 2) * NS + s < N_CHUNKS)
            def _():
                wait_idx(i + 2, nj)
                xform(nj)
                issue_gather(nj, bufs[nj])

            @pl.when(valid)
            def _():
                scale(jj, buf)
                pltpu.sync_copy(buf, acc_sh.at[dst_v.at[jj]], add=True)

            @pl.when((i + 3) * NS + s < N_CHUNKS)
            def _():
                issue_idx(i + 3, jj)
        return carry

    lax.fori_loop(0, (CHUNKS_PER_TILE + 2) // 3, ring_body, 0)
    plsc.subcore_barrier()
    pltpu.sync_copy(acc_sh.at[pl.ds(start, ROWS_MAIN)],
                    out_hbm.at[c, pl.ds(start, ROWS_MAIN)])

    @pl.when(s == NS - 1)
    def _():
        pltpu.sync_copy(
            acc_sh.at[pl.ds(NS * ROWS_MAIN, N_NODES - NS * ROWS_MAIN)],
            out_hbm.at[c, pl.ds(NS * ROWS_MAIN, N_NODES - NS * ROWS_MAIN)])


_sc_agg = functools.partial(
    pl.kernel,
    out_type=jax.ShapeDtypeStruct((NC, N_NODES, HALF), jnp.float32),
    mesh=plsc.VectorSubcoreMesh(core_axis_name="c", subcore_axis_name="s"),
    scratch_types=[
        pltpu.VMEM((3, CHUNK), jnp.int32),
        pltpu.VMEM((3, CHUNK), jnp.int32),
        pltpu.VMEM((3, CHUNK), jnp.float32),
        pltpu.VMEM((CHUNK, HALF), jnp.float32),
        pltpu.VMEM((CHUNK, HALF), jnp.float32),
        pltpu.VMEM((CHUNK, HALF), jnp.float32),
        pltpu.VMEM_SHARED((N_NODES, HALF), jnp.float32),
        pltpu.SemaphoreType.DMA,
        pltpu.SemaphoreType.DMA,
    ],
)(_sc_agg_body)


def _mm_body(a0_ref, a1_ref, w0_ref, w1_ref, o_ref):
    acc = jnp.dot(a0_ref[...], w0_ref[...],
                  preferred_element_type=jnp.float32,
                  precision=lax.Precision.HIGHEST)
    acc = acc + jnp.dot(a1_ref[...], w1_ref[...],
                        preferred_element_type=jnp.float32,
                        precision=lax.Precision.HIGHEST)
    o_ref[...] = jnp.maximum(acc, 0.0)


M_BLK = 1000


def _mm_relu(agg2, w):
    return pl.pallas_call(
        _mm_body,
        grid=(N_NODES // M_BLK,),
        in_specs=[
            pl.BlockSpec((M_BLK, HALF), lambda i: (i, 0)),
            pl.BlockSpec((M_BLK, HALF), lambda i: (i, 0)),
            pl.BlockSpec((HALF, D_OUT), lambda i: (0, 0)),
            pl.BlockSpec((HALF, D_OUT), lambda i: (0, 0)),
        ],
        out_specs=pl.BlockSpec((M_BLK, D_OUT), lambda i: (i, 0)),
        out_shape=jax.ShapeDtypeStruct((N_NODES, D_OUT), jnp.float32),
    )(agg2[0], agg2[1], w[:HALF], w[HALF:])


def kernel(x, edge_index, edge_values, W):
    x2 = x.reshape(2 * N_NODES, HALF)
    agg2 = _sc_agg(x2, edge_index, edge_values)
    return _mm_relu(agg2, W)
